# Initial kernel scaffold; baseline (speedup 1.0000x reference)
import jax
import jax.numpy as jnp
from jax.experimental import pallas as pl


def kernel(x, emb_table, pos_table):
    raise NotImplementedError("write your pallas kernel here")


# speedup vs baseline: 4.2249x; 4.2249x over previous
import functools

import jax
import jax.numpy as jnp
from jax import lax
from jax.experimental import pallas as pl
from jax.experimental.pallas import tpu as pltpu
from jax.experimental.pallas import tpu_sc as plsc

_BATCH = 1024
_SEQ = 200
_HID = 128
_ROWS = _BATCH * _SEQ
_NW = 32
_ROWS_PER_W = _ROWS // _NW
_SEQS_PER_W = _ROWS_PER_W // _SEQ
_LANES = 16

_mesh = plsc.VectorSubcoreMesh(core_axis_name="c", subcore_axis_name="s")


@functools.partial(
    pl.kernel,
    mesh=_mesh,
    out_type=jax.ShapeDtypeStruct((_ROWS, _HID), jnp.float32),
    scratch_types=[
        pltpu.VMEM((_ROWS_PER_W,), jnp.int32),
        pltpu.VMEM((_SEQ, _HID), jnp.float32),
        pltpu.VMEM((_SEQ, _HID), jnp.float32),
        pltpu.SemaphoreType.DMA,
    ],
)
def _emb_lookup(x_hbm, emb_hbm, pos_hbm, out_hbm, idx_v, pos_v, tok_v, sem):
    wid = lax.axis_index("s") * 2 + lax.axis_index("c")
    base = wid * _ROWS_PER_W
    pltpu.sync_copy(x_hbm.at[pl.ds(base, _ROWS_PER_W)], idx_v)
    pltpu.sync_copy(pos_hbm, pos_v)

    def seq_body(s, carry):
        pltpu.async_copy(
            emb_hbm.at[idx_v.at[pl.ds(s * _SEQ, _SEQ)]], tok_v, sem
        ).wait()

        def add_row(l, c):
            for j in range(_HID // _LANES):
                sl = pl.ds(j * _LANES, _LANES)
                tok_v[l, sl] = tok_v[l, sl] + pos_v[l, sl]
            return c

        lax.fori_loop(0, _SEQ, add_row, 0)
        pltpu.sync_copy(tok_v, out_hbm.at[pl.ds(base + s * _SEQ, _SEQ)])
        return carry

    lax.fori_loop(0, _SEQS_PER_W, seq_body, 0)


def kernel(x, emb_table, pos_table):
    xf = x.reshape(-1).astype(jnp.int32)
    out = _emb_lookup(xf, emb_table, pos_table)
    return out.reshape(_BATCH, _SEQ, _HID)
